# Initial kernel scaffold; baseline (speedup 1.0000x reference)
import jax
import jax.numpy as jnp
from jax.experimental import pallas as pl


def kernel(x, durations, max_len):
    raise NotImplementedError("write your pallas kernel here")


# speedup vs baseline: 3.2186x; 3.2186x over previous
import functools

import jax
import jax.numpy as jnp
from jax import lax
from jax.experimental import pallas as pl
from jax.experimental.pallas import tpu as pltpu
from jax.experimental.pallas import tpu_sc as plsc

_T = 2048
_LANES = 16


@functools.lru_cache(maxsize=None)
def _lr_kernel(B, L, D):
    T = _T
    NC = 2
    n_half = T // NC
    n_chunk = 128
    chunks_per_half = n_half // n_chunk
    rows_per_chunk = n_chunk // _LANES
    zero_row = B * L
    mesh = plsc.VectorSubcoreMesh(core_axis_name="c", subcore_axis_name="s")

    @functools.partial(
        pl.kernel,
        out_type=jax.ShapeDtypeStruct((B, T, D), jnp.float32),
        mesh=mesh,
        compiler_params=pltpu.CompilerParams(needs_layout_passes=False),
        scratch_types=[
            pltpu.VMEM((L,), jnp.float32),
            pltpu.VMEM((T,), jnp.int32),
            pltpu.VMEM((T // n_chunk, n_chunk), jnp.int32),
            pltpu.VMEM((_LANES,), jnp.int32),
            pltpu.VMEM((n_chunk, D), jnp.float32),
            pltpu.SemaphoreType.DMA,
        ],
    )
    def k(table_hbm, dur_hbm, off_hbm, out_hbm,
          dur_v, seg_v, idx_v, off_v, buf_v, sem):
        b = lax.axis_index("s")
        h = lax.axis_index("c")
        pltpu.sync_copy(dur_hbm.at[b], dur_v)
        pltpu.sync_copy(off_hbm, off_v)
        off = off_v[...]

        def zero_body(i, _):
            seg_v[pl.ds(i * _LANES, _LANES)] = jnp.zeros((_LANES,), jnp.int32)
            return 0

        lax.fori_loop(0, T // _LANES, zero_body, 0)

        def scat_body(i, carry):
            tot, basev = carry
            dv = jnp.maximum(dur_v[pl.ds(i * _LANES, _LANES)], 0.0)
            di = (dv + 0.5).astype(jnp.int32)
            cum = plsc.cumsum(di) + tot
            pos = cum - di - off
            lv = lax.iota(jnp.int32, _LANES) + i * _LANES + 1
            valid = di > 0
            m = valid & (pos >= 0) & (pos < T)
            plsc.store_scatter(seg_v, [jnp.clip(pos, 0, T - 1)], lv, mask=m)
            basev = jnp.maximum(basev, jnp.where(valid & (pos < 0), lv, 0))
            return jnp.max(cum), basev

        total, basev = lax.fori_loop(
            0, L // _LANES, scat_body,
            (jnp.asarray(0, jnp.int32), jnp.zeros((_LANES,), jnp.int32)))
        base = jnp.max(basev)

        def cm_body(i, mc):
            s = seg_v[pl.ds(i * _LANES, _LANES)]
            cm = jnp.maximum(plsc.cummax(s), mc)
            idx = jnp.clip(cm - 1, 0, L - 1)
            kv = lax.iota(jnp.int32, _LANES) + i * _LANES
            gidx = jnp.where(kv + off < total, b * L + idx, zero_row)
            idx_v[i // rows_per_chunk,
                  pl.ds((i % rows_per_chunk) * _LANES, _LANES)] = gidx
            return jnp.max(cm)

        lax.fori_loop(0, T // _LANES, cm_body, base)

        for cix in range(chunks_per_half):
            r = h * chunks_per_half + cix
            pltpu.async_copy(table_hbm.at[idx_v.at[r]], buf_v, sem).wait()
            pltpu.sync_copy(buf_v, out_hbm.at[b, pl.ds(r * n_chunk, n_chunk), :])

    return k


def kernel(x, durations, max_len):
    B, L, D = x.shape
    table = jnp.concatenate(
        [x.reshape(B * L, D), jnp.zeros((1, D), x.dtype)], axis=0)
    off = jnp.full((_LANES,), jnp.asarray(max_len, jnp.int32) - _T, jnp.int32)
    return _lr_kernel(B, L, D)(table, durations, off)


# speedup vs baseline: 3.3145x; 1.0298x over previous
import functools

import jax
import jax.numpy as jnp
from jax import lax
from jax.experimental import pallas as pl
from jax.experimental.pallas import tpu as pltpu
from jax.experimental.pallas import tpu_sc as plsc

_T = 2048
_LANES = 16


@functools.lru_cache(maxsize=None)
def _lr_kernel(B, L, D):
    T = _T
    NC = 2
    n_half = T // NC
    n_chunk = 128
    chunks_per_half = n_half // n_chunk
    rows_per_chunk = n_chunk // _LANES
    zero_row = B * L
    mesh = plsc.VectorSubcoreMesh(core_axis_name="c", subcore_axis_name="s")

    @functools.partial(
        pl.kernel,
        out_type=jax.ShapeDtypeStruct((B, T, D), jnp.float32),
        mesh=mesh,
        compiler_params=pltpu.CompilerParams(needs_layout_passes=False),
        scratch_types=[
            pltpu.VMEM((L,), jnp.float32),
            pltpu.VMEM((T,), jnp.int32),
            pltpu.VMEM((T // n_chunk, n_chunk), jnp.int32),
            pltpu.VMEM((_LANES,), jnp.int32),
            pltpu.VMEM((n_chunk, D), jnp.float32),
            pltpu.VMEM((n_chunk, D), jnp.float32),
            pltpu.SemaphoreType.DMA,
            pltpu.SemaphoreType.DMA,
            pltpu.SemaphoreType.DMA,
            pltpu.SemaphoreType.DMA,
        ],
    )
    def k(table_hbm, dur_hbm, off_hbm, out_hbm,
          dur_v, seg_v, idx_v, off_v, buf_a, buf_b, gs_a, gs_b, ws_a, ws_b):
        b = lax.axis_index("s")
        h = lax.axis_index("c")
        pltpu.sync_copy(dur_hbm.at[b], dur_v)
        pltpu.sync_copy(off_hbm, off_v)
        off = off_v[...]

        def zero_body(i, _):
            seg_v[pl.ds(i * _LANES, _LANES)] = jnp.zeros((_LANES,), jnp.int32)
            return 0

        lax.fori_loop(0, T // _LANES, zero_body, 0)

        def scat_body(i, carry):
            tot, basev = carry
            dv = jnp.maximum(dur_v[pl.ds(i * _LANES, _LANES)], 0.0)
            di = (dv + 0.5).astype(jnp.int32)
            cum = plsc.cumsum(di) + tot
            pos = cum - di - off
            lv = lax.iota(jnp.int32, _LANES) + i * _LANES + 1
            valid = di > 0
            m = valid & (pos >= 0) & (pos < T)
            plsc.store_scatter(seg_v, [jnp.clip(pos, 0, T - 1)], lv, mask=m)
            basev = jnp.maximum(basev, jnp.where(valid & (pos < 0), lv, 0))
            return jnp.max(cum), basev

        total, basev = lax.fori_loop(
            0, L // _LANES, scat_body,
            (jnp.asarray(0, jnp.int32), jnp.zeros((_LANES,), jnp.int32)))
        base = jnp.max(basev)

        def cm_body(i, mc):
            s = seg_v[pl.ds(i * _LANES, _LANES)]
            cm = jnp.maximum(plsc.cummax(s), mc)
            idx = jnp.clip(cm - 1, 0, L - 1)
            kv = lax.iota(jnp.int32, _LANES) + i * _LANES
            gidx = jnp.where(kv + off < total, b * L + idx, zero_row)
            idx_v[i // rows_per_chunk,
                  pl.ds((i % rows_per_chunk) * _LANES, _LANES)] = gidx
            return jnp.max(cm)

        lax.fori_loop(0, T // _LANES, cm_body, base)

        bufs = (buf_a, buf_b)
        gsems = (gs_a, gs_b)
        wsems = (ws_a, ws_b)

        def chunk_row(cix):
            return 2 * cix + h

        gathers = [None, None]
        writes = [None, None]
        gathers[0] = pltpu.async_copy(
            table_hbm.at[idx_v.at[chunk_row(0)]], bufs[0], gsems[0])
        for cix in range(chunks_per_half):
            pb = cix % 2
            gathers[pb].wait()
            r = chunk_row(cix)
            writes[pb] = pltpu.async_copy(
                bufs[pb], out_hbm.at[b, pl.ds(r * n_chunk, n_chunk), :],
                wsems[pb])
            if cix + 1 < chunks_per_half:
                nb = (cix + 1) % 2
                if writes[nb] is not None:
                    writes[nb].wait()
                gathers[nb] = pltpu.async_copy(
                    table_hbm.at[idx_v.at[chunk_row(cix + 1)]], bufs[nb],
                    gsems[nb])
        writes[0].wait()
        writes[1].wait()

    return k


def kernel(x, durations, max_len):
    B, L, D = x.shape
    table = jnp.concatenate(
        [x.reshape(B * L, D), jnp.zeros((1, D), x.dtype)], axis=0)
    off = jnp.full((_LANES,), jnp.asarray(max_len, jnp.int32) - _T, jnp.int32)
    return _lr_kernel(B, L, D)(table, durations, off)
